# baseline (speedup 1.0000x reference)
import functools

import jax
import jax.numpy as jnp
from jax import lax
from jax.experimental import pallas as pl
from jax.experimental.pallas import tpu as pltpu
from jax.experimental.pallas import tpu_sc as plsc

HIDDEN = 64
VOCAB = 1000000
B, L = 1024, 200
ROWS = B * L
NW = 32
RPW = ROWS // NW
CH = 64
NCH = RPW // CH
NB = 5
NBLK = NCH // NB
BB = 32
TBLK = 512


def _tr_body(x_ref, o_ref):
    x = x_ref[...]
    xt = jnp.transpose(x, (1, 0))
    o_ref[...] = jnp.concatenate(
        [xt, jnp.zeros((TBLK, 128 - HIDDEN), jnp.float32)], axis=-1)


def _tc_transpose(wt_t):
    grid = (VOCAB + TBLK - 1) // TBLK
    return pl.pallas_call(
        _tr_body,
        grid=(grid,),
        in_specs=[pl.BlockSpec((HIDDEN, TBLK), lambda i: (0, i))],
        out_specs=pl.BlockSpec((TBLK, 128), lambda i: (i, 0)),
        out_shape=jax.ShapeDtypeStruct((VOCAB, 128), jnp.float32),
    )(wt_t)


@functools.cache
def _make_sc_gather():
    mesh = plsc.VectorSubcoreMesh(core_axis_name="c", subcore_axis_name="s")

    @functools.partial(
        pl.kernel,
        mesh=mesh,
        out_type=jax.ShapeDtypeStruct((ROWS, 128), jnp.float32),
        scratch_types=[
            pltpu.VMEM((NCH, CH), jnp.int32),
            pltpu.VMEM((2, NB, CH, 128), jnp.float32),
            pltpu.SemaphoreType.DMA((2, NB)),
            pltpu.SemaphoreType.DMA((2, NB)),
        ],
    )
    def gather_k(ids_hbm, table_hbm, out_hbm, idx_v, rows_v, gsems, wsems):
        wid = lax.axis_index("s") * 2 + lax.axis_index("c")
        pltpu.sync_copy(ids_hbm.at[wid], idx_v)
        base = wid * RPW

        def fire(j, s, b):
            return pltpu.async_copy(
                table_hbm.at[idx_v.at[j]], rows_v.at[s, b], gsems.at[s, b])

        def write(j, s, b):
            pltpu.async_copy(
                rows_v.at[s, b], out_hbm.at[pl.ds(base + j * CH, CH)],
                wsems.at[s, b])

        def wait_write(s, b):
            pltpu.make_async_copy(
                rows_v.at[s, b], out_hbm.at[pl.ds(base, CH)],
                wsems.at[s, b]).wait()

        def do_block(k, s, reuse):
            if reuse:
                for b in range(NB):
                    wait_write(s, b)
            copies = [fire(k * NB + b, s, b) for b in range(NB)]
            for b in range(NB):
                copies[b].wait()
                write(k * NB + b, s, b)

        do_block(0, 0, False)
        do_block(1, 1, False)

        def body(i, carry):
            k = 2 * i + 2
            do_block(k, 0, True)
            do_block(k + 1, 1, True)
            return carry

        lax.fori_loop(0, (NBLK - 2) // 2, body, 0)

        for s in range(2):
            for b in range(NB):
                wait_write(s, b)

    return gather_k


def _ln_body(x_ref, pos_ref, g_ref, b_ref, o_ref):
    x = x_ref[...][..., :HIDDEN] + pos_ref[...][None, :, :]
    mean = jnp.mean(x, axis=-1, keepdims=True)
    var = jnp.mean(jnp.square(x - mean), axis=-1, keepdims=True)
    y = (x - mean) * lax.rsqrt(var + 1e-5)
    o_ref[...] = y * g_ref[...][None, :, :] + b_ref[...][None, :, :]


def _tc_ln(x3, pos, gamma, beta):
    return pl.pallas_call(
        _ln_body,
        grid=(B // BB,),
        in_specs=[
            pl.BlockSpec((BB, L, 128), lambda i: (i, 0, 0)),
            pl.BlockSpec((L, HIDDEN), lambda i: (0, 0)),
            pl.BlockSpec((1, HIDDEN), lambda i: (0, 0)),
            pl.BlockSpec((1, HIDDEN), lambda i: (0, 0)),
        ],
        out_specs=pl.BlockSpec((BB, L, HIDDEN), lambda i: (i, 0, 0)),
        out_shape=jax.ShapeDtypeStruct((B, L, HIDDEN), jnp.float32),
    )(x3, pos, gamma, beta)


def kernel(input_ids, word_table, pos_table, ln_gamma, ln_beta):
    ids = input_ids.astype(jnp.int32).reshape(NW, NCH, CH)
    tab128 = _tc_transpose(word_table.T)
    gathered = _make_sc_gather()(ids, tab128)
    x3 = gathered.reshape(B, L, 128)
    pos = pos_table[:L]
    return _tc_ln(x3, pos, ln_gamma.reshape(1, HIDDEN), ln_beta.reshape(1, HIDDEN))


# speedup vs baseline: 2.0458x; 2.0458x over previous
import functools

import jax
import jax.numpy as jnp
from jax import lax
from jax.experimental import pallas as pl
from jax.experimental.pallas import tpu as pltpu
from jax.experimental.pallas import tpu_sc as plsc

HIDDEN = 64
VOCAB = 1000000
B, L = 1024, 200
ROWS = B * L
NW = 32
RPW = ROWS // NW
CH = 64
NCH = RPW // CH
NB = 5
NBLK = NCH // NB
BB = 32
TBLK = 2048


def _tr_body(x_ref, o_ref):
    x = x_ref[...]
    xt = jnp.transpose(x, (1, 0))
    o_ref[...] = jnp.concatenate(
        [xt, jnp.zeros((TBLK, 128 - HIDDEN), jnp.float32)], axis=-1)


def _tc_transpose(wt_t):
    grid = (VOCAB + TBLK - 1) // TBLK
    return pl.pallas_call(
        _tr_body,
        grid=(grid,),
        in_specs=[pl.BlockSpec((HIDDEN, TBLK), lambda i: (0, i))],
        out_specs=pl.BlockSpec((TBLK, 128), lambda i: (i, 0)),
        out_shape=jax.ShapeDtypeStruct((VOCAB, 128), jnp.float32),
    )(wt_t)


@functools.cache
def _make_sc_gather():
    mesh = plsc.VectorSubcoreMesh(core_axis_name="c", subcore_axis_name="s")

    @functools.partial(
        pl.kernel,
        mesh=mesh,
        out_type=jax.ShapeDtypeStruct((ROWS, 128), jnp.float32),
        scratch_types=[
            pltpu.VMEM((NCH, CH), jnp.int32),
            pltpu.VMEM((2, NB, CH, 128), jnp.float32),
            pltpu.SemaphoreType.DMA((2, NB)),
            pltpu.SemaphoreType.DMA((2, NB)),
        ],
    )
    def gather_k(ids_hbm, table_hbm, out_hbm, idx_v, rows_v, gsems, wsems):
        wid = lax.axis_index("s") * 2 + lax.axis_index("c")
        pltpu.sync_copy(ids_hbm.at[wid], idx_v)
        base = wid * RPW

        def fire(j, s, b):
            return pltpu.async_copy(
                table_hbm.at[idx_v.at[j]], rows_v.at[s, b], gsems.at[s, b])

        def write(j, s, b):
            pltpu.async_copy(
                rows_v.at[s, b], out_hbm.at[pl.ds(base + j * CH, CH)],
                wsems.at[s, b])

        def wait_write(s, b):
            pltpu.make_async_copy(
                rows_v.at[s, b], out_hbm.at[pl.ds(base, CH)],
                wsems.at[s, b]).wait()

        def do_block(k, s, reuse):
            if reuse:
                for b in range(NB):
                    wait_write(s, b)
            copies = [fire(k * NB + b, s, b) for b in range(NB)]
            for b in range(NB):
                copies[b].wait()
                write(k * NB + b, s, b)

        do_block(0, 0, False)
        do_block(1, 1, False)

        def body(i, carry):
            k = 2 * i + 2
            do_block(k, 0, True)
            do_block(k + 1, 1, True)
            return carry

        lax.fori_loop(0, (NBLK - 2) // 2, body, 0)

        for s in range(2):
            for b in range(NB):
                wait_write(s, b)

    return gather_k


def _ln_body(x_ref, pos_ref, g_ref, b_ref, o_ref):
    x = x_ref[...][..., :HIDDEN] + pos_ref[...][None, :, :]
    mean = jnp.mean(x, axis=-1, keepdims=True)
    var = jnp.mean(jnp.square(x - mean), axis=-1, keepdims=True)
    y = (x - mean) * lax.rsqrt(var + 1e-5)
    o_ref[...] = y * g_ref[...][None, :, :] + b_ref[...][None, :, :]


def _tc_ln(x3, pos, gamma, beta):
    return pl.pallas_call(
        _ln_body,
        grid=(B // BB,),
        in_specs=[
            pl.BlockSpec((BB, L, 128), lambda i: (i, 0, 0)),
            pl.BlockSpec((L, HIDDEN), lambda i: (0, 0)),
            pl.BlockSpec((1, HIDDEN), lambda i: (0, 0)),
            pl.BlockSpec((1, HIDDEN), lambda i: (0, 0)),
        ],
        out_specs=pl.BlockSpec((BB, L, HIDDEN), lambda i: (i, 0, 0)),
        out_shape=jax.ShapeDtypeStruct((B, L, HIDDEN), jnp.float32),
    )(x3, pos, gamma, beta)


def kernel(input_ids, word_table, pos_table, ln_gamma, ln_beta):
    ids = input_ids.astype(jnp.int32).reshape(NW, NCH, CH)
    tab128 = _tc_transpose(word_table.T)
    gathered = _make_sc_gather()(ids, tab128)
    x3 = gathered.reshape(B, L, 128)
    pos = pos_table[:L]
    return _tc_ln(x3, pos, ln_gamma.reshape(1, HIDDEN), ln_beta.reshape(1, HIDDEN))
